# baseline (speedup 1.0000x reference)
import functools

import jax
import jax.numpy as jnp
from jax import lax
from jax.experimental import pallas as pl
from jax.experimental.pallas import tpu as pltpu
from jax.experimental.pallas import tpu_sc as plsc

EMBED_DIM = 32
NUM_CORES = 2
NUM_SUBCORES = 16
NUM_WORKERS = NUM_CORES * NUM_SUBCORES
ETILE = EMBED_DIM // 8


@functools.partial(jax.jit, static_argnames=("batch", "hist"))
def _gather_rows(idx_t, table, batch, hist):
    bw = batch // NUM_WORKERS
    jt = bw // 128

    mesh = plsc.VectorSubcoreMesh(core_axis_name="c", subcore_axis_name="s")

    @functools.partial(
        pl.kernel,
        mesh=mesh,
        out_type=jax.ShapeDtypeStruct((hist, ETILE, batch // 128, 8, 128), jnp.float32),
        scratch_types=[
            [pltpu.VMEM((bw,), jnp.int32)] * 2,
            [pltpu.VMEM((bw, EMBED_DIM), jnp.float32)] * 2,
            [pltpu.VMEM((ETILE, jt, 8, 128), jnp.float32)] * 2,
            [pltpu.SemaphoreType.DMA] * 2,
            [pltpu.SemaphoreType.DMA] * 2,
            [pltpu.SemaphoreType.DMA] * 2,
        ],
        compiler_params=pltpu.CompilerParams(
            use_tc_tiling_on_sc=False, needs_layout_passes=False
        ),
    )
    def k(idx_hbm, table_hbm, out_hbm, idx_v, rows_v, rows_t, isem, gsem, osem):
        wid = lax.axis_index("s") * NUM_CORES + lax.axis_index("c")
        b0 = wid * bw
        iota16 = lax.iota(jnp.int32, 16)

        def idx_load(t, p):
            return pltpu.make_async_copy(
                idx_hbm.at[t, pl.ds(b0, bw)], idx_v[p], isem[p]
            )

        def gath(p):
            return pltpu.make_async_copy(table_hbm.at[idx_v[p]], rows_v[p], gsem[p])

        def store(t, p):
            return pltpu.make_async_copy(
                rows_t[p], out_hbm.at[t, :, pl.ds(wid * jt, jt)], osem[p]
            )

        def transpose(p):
            rv, rt = rows_v[p], rows_t[p]

            def trans_jb(jb, carry):
                rows16 = iota16 + jb * 16
                jo = jb // 8
                bo = (jb % 8) * 16
                for g in range(EMBED_DIM // 16):
                    vs = [
                        plsc.load_gather(rv, [rows16, iota16 * 0 + (16 * g + i)])
                        for i in range(16)
                    ]
                    for i in range(16):
                        e = 16 * g + i
                        rt[e // 8, jo, e % 8, pl.ds(bo, 16)] = vs[i]
                return carry

            lax.fori_loop(0, bw // 16, trans_jb, 0)

        def half(t, p, first, last):
            gath(p).wait()
            if not last:
                idx_load(t + 2, p).start()
            q = 1 - p
            idx_load(t + 1, q).wait()
            gath(q).start()
            if not first:
                store(t, p).wait()
            transpose(p)
            store(t, p).start()

        idx_load(0, 0).start()
        idx_load(0, 0).wait()
        gath(0).start()
        idx_load(1, 1).start()

        def pair(g, carry):
            t0 = 2 * g

            @pl.when(g == 0)
            def _():
                half(t0, 0, True, False)
                half(t0 + 1, 1, True, False)

            @pl.when(g > 0)
            def _():
                half(t0, 0, False, False)
                half(t0 + 1, 1, False, False)

            return carry

        lax.fori_loop(0, hist // 2 - 1, pair, 0)

        tl = hist - 2
        gath(0).wait()
        idx_load(tl + 1, 1).wait()
        gath(1).start()
        store(tl, 0).wait()
        transpose(0)
        store(tl, 0).start()
        gath(1).wait()
        store(tl + 1, 1).wait()
        transpose(1)
        store(tl + 1, 1).start()
        store(tl, 0).wait()
        store(tl + 1, 1).wait()

    return k(idx_t, table)


def kernel(inputs, table):
    batch, hist = inputs.shape
    idx_t = inputs.T.astype(jnp.int32)
    x5 = _gather_rows(idx_t, table, batch, hist)
    z = jnp.transpose(x5, (0, 1, 3, 2, 4)).reshape(hist, EMBED_DIM, batch)
    return jnp.transpose(z, (2, 0, 1))


# speedup vs baseline: 1.4401x; 1.4401x over previous
import functools

import jax
import jax.numpy as jnp
from jax import lax
from jax.experimental import pallas as pl
from jax.experimental.pallas import tpu as pltpu
from jax.experimental.pallas import tpu_sc as plsc

EMBED_DIM = 32
NUM_CORES = 2
NUM_SUBCORES = 16
NUM_WORKERS = NUM_CORES * NUM_SUBCORES
ETILE = EMBED_DIM // 8


@functools.partial(jax.jit, static_argnames=("batch", "hist"))
def _gather_rows(idx_t, table, batch, hist):
    bw = batch // NUM_WORKERS
    jt = bw // 128

    mesh = plsc.VectorSubcoreMesh(core_axis_name="c", subcore_axis_name="s")

    @functools.partial(
        pl.kernel,
        mesh=mesh,
        out_type=jax.ShapeDtypeStruct((hist, ETILE, batch // 128, 8, 128), jnp.float32),
        scratch_types=[
            [pltpu.VMEM((bw,), jnp.int32)] * 2,
            [pltpu.VMEM((bw, EMBED_DIM), jnp.float32)] * 2,
            [pltpu.VMEM((ETILE * jt * 8, 128), jnp.float32)] * 2,
            [pltpu.SemaphoreType.DMA] * 2,
            [pltpu.SemaphoreType.DMA] * 2,
            [pltpu.SemaphoreType.DMA] * 2,
        ],
        compiler_params=pltpu.CompilerParams(
            use_tc_tiling_on_sc=False, needs_layout_passes=False
        ),
    )
    def k(idx_hbm, table_hbm, out_hbm, idx_v, rows_v, rows_t, isem, gsem, osem):
        wid = lax.axis_index("s") * NUM_CORES + lax.axis_index("c")
        b0 = wid * bw
        iota16 = lax.iota(jnp.int32, 16)

        def idx_load(t, p):
            return pltpu.make_async_copy(
                idx_hbm.at[t, pl.ds(b0, bw)], idx_v[p], isem[p]
            )

        def gath(p):
            return pltpu.make_async_copy(table_hbm.at[idx_v[p]], rows_v[p], gsem[p])

        def store_copies(t, p):
            return [
                pltpu.make_async_copy(
                    rows_t[p].at[pl.ds(E * jt * 8 + Jo * 8, 8), :],
                    out_hbm.at[t, E, wid * jt + Jo],
                    osem[p],
                )
                for E in range(ETILE)
                for Jo in range(jt)
            ]

        dd = [(iota16 + k) & 15 for k in range(16)]
        lut2 = [((d >> 3) << 5) | (d & 7) for d in dd]

        def transpose(p):
            rv, rt = rows_v[p], rows_t[p]

            def trans_jb(jb2, carry):
                row16 = iota16 + jb2 * 16
                jo = jb2 // 8
                colidx = iota16 + (jb2 % 8) * 16
                for e0 in (0, 16):
                    rbase = e0 * ETILE + jo * 8
                    for k in range(16):
                        coll = dd[k] if e0 == 0 else dd[k] | 16
                        v = plsc.load_gather(rv, [row16, coll])
                        plsc.store_scatter(rt, [lut2[k] + rbase, colidx], v)
                return carry

            lax.fori_loop(0, bw // 16, trans_jb, 0)

        def half(t, p, first, last):
            gath(p).wait()
            if not last:
                idx_load(t + 2, p).start()
            q = 1 - p
            idx_load(t + 1, q).wait()
            gath(q).start()
            if not first:
                for c in store_copies(t, p):
                    c.wait()
            transpose(p)
            for c in store_copies(t, p):
                c.start()

        idx_load(0, 0).start()
        idx_load(0, 0).wait()
        gath(0).start()
        idx_load(1, 1).start()

        def pair(g, carry):
            t0 = 2 * g

            @pl.when(g == 0)
            def _():
                half(t0, 0, True, False)
                half(t0 + 1, 1, True, False)

            @pl.when(g > 0)
            def _():
                half(t0, 0, False, False)
                half(t0 + 1, 1, False, False)

            return carry

        lax.fori_loop(0, hist // 2 - 1, pair, 0)

        tl = hist - 2
        gath(0).wait()
        idx_load(tl + 1, 1).wait()
        gath(1).start()
        for c in store_copies(tl, 0):
            c.wait()
        transpose(0)
        for c in store_copies(tl, 0):
            c.start()
        gath(1).wait()
        for c in store_copies(tl + 1, 1):
            c.wait()
        transpose(1)
        for c in store_copies(tl + 1, 1):
            c.start()
        for c in store_copies(tl, 0):
            c.wait()
        for c in store_copies(tl + 1, 1):
            c.wait()

    return k(idx_t, table)


def kernel(inputs, table):
    batch, hist = inputs.shape
    idx_t = inputs.T.astype(jnp.int32)
    x5 = _gather_rows(idx_t, table, batch, hist)
    z = jnp.transpose(x5, (0, 1, 3, 2, 4)).reshape(hist, EMBED_DIM, batch)
    return jnp.transpose(z, (2, 0, 1))


# speedup vs baseline: 2.2478x; 1.5608x over previous
import functools

import jax
import jax.numpy as jnp
from jax import lax
from jax.experimental import pallas as pl
from jax.experimental.pallas import tpu as pltpu
from jax.experimental.pallas import tpu_sc as plsc

EMBED_DIM = 32
NUM_CORES = 2
NUM_SUBCORES = 16
NUM_WORKERS = NUM_CORES * NUM_SUBCORES
ETILE = EMBED_DIM // 8


@functools.partial(jax.jit, static_argnames=("batch", "hist"))
def _gather_rows(idx_t, table, batch, hist):
    bw = batch // NUM_WORKERS
    jt = bw // 128

    mesh = plsc.VectorSubcoreMesh(core_axis_name="c", subcore_axis_name="s")

    @functools.partial(
        pl.kernel,
        mesh=mesh,
        out_type=jax.ShapeDtypeStruct((hist, ETILE, batch // 128, 8, 128), jnp.float32),
        scratch_types=[
            [pltpu.VMEM((bw,), jnp.int32)] * 2,
            [pltpu.VMEM((bw, EMBED_DIM), jnp.float32)] * 2,
            [pltpu.VMEM((ETILE * jt * 8, 128), jnp.float32)] * 2,
            [pltpu.SemaphoreType.DMA] * 2,
            [pltpu.SemaphoreType.DMA] * 2,
            [pltpu.SemaphoreType.DMA] * 2,
        ],
        compiler_params=pltpu.CompilerParams(
            use_tc_tiling_on_sc=False, needs_layout_passes=False
        ),
    )
    def k(idx_hbm, table_hbm, out_hbm, idx_v, rows_v, rows_t, isem, gsem, osem):
        wid = lax.axis_index("s") * NUM_CORES + lax.axis_index("c")
        b0 = wid * bw
        iota16 = lax.iota(jnp.int32, 16)

        def idx_load(t, p):
            return pltpu.make_async_copy(
                idx_hbm.at[t, pl.ds(b0, bw)], idx_v[p], isem[p]
            )

        def gath(p):
            return pltpu.make_async_copy(table_hbm.at[idx_v[p]], rows_v[p], gsem[p])

        def store_copies(t, p):
            return [
                pltpu.make_async_copy(
                    rows_t[p].at[pl.ds(E * jt * 8 + Jo * 8, 8), :],
                    out_hbm.at[t, E, wid * jt + Jo],
                    osem[p],
                )
                for E in range(ETILE)
                for Jo in range(jt)
            ]

        dd = [(iota16 + k) & 15 for k in range(16)]
        lut2 = [((d >> 3) << 5) | (d & 7) for d in dd]

        def transpose(p):
            rv, rt = rows_v[p], rows_t[p]

            def trans_jb(jb2, carry):
                row16 = iota16 + jb2 * 16
                jo = jb2 // 8
                colidx = iota16 + (jb2 % 8) * 16
                for e0 in (0, 16):
                    rbase = e0 * ETILE + jo * 8
                    for k0 in range(0, 16, 4):
                        vs = [
                            plsc.load_gather(
                                rv,
                                [row16, dd[k0 + i] if e0 == 0 else dd[k0 + i] | 16],
                            )
                            for i in range(4)
                        ]
                        for i in range(4):
                            plsc.store_scatter(
                                rt, [lut2[k0 + i] + rbase, colidx], vs[i]
                            )
                return carry

            lax.fori_loop(0, bw // 16, trans_jb, 0)

        def half(t, p, first, last):
            gath(p).wait()
            if not last:
                idx_load(t + 2, p).start()
            q = 1 - p
            idx_load(t + 1, q).wait()
            gath(q).start()
            if not first:
                for c in store_copies(t, p):
                    c.wait()
            transpose(p)
            for c in store_copies(t, p):
                c.start()

        idx_load(0, 0).start()
        idx_load(0, 0).wait()
        gath(0).start()
        idx_load(1, 1).start()

        def pair(g, carry):
            t0 = 2 * g

            @pl.when(g == 0)
            def _():
                half(t0, 0, True, False)
                half(t0 + 1, 1, True, False)

            @pl.when(g > 0)
            def _():
                half(t0, 0, False, False)
                half(t0 + 1, 1, False, False)

            return carry

        lax.fori_loop(0, hist // 2 - 1, pair, 0)

        tl = hist - 2
        gath(0).wait()
        idx_load(tl + 1, 1).wait()
        gath(1).start()
        for c in store_copies(tl, 0):
            c.wait()
        transpose(0)
        for c in store_copies(tl, 0):
            c.start()
        gath(1).wait()
        for c in store_copies(tl + 1, 1):
            c.wait()
        transpose(1)
        for c in store_copies(tl + 1, 1):
            c.start()
        for c in store_copies(tl, 0):
            c.wait()
        for c in store_copies(tl + 1, 1):
            c.wait()

    return k(idx_t, table)


def kernel(inputs, table):
    batch, hist = inputs.shape
    idx_t = inputs.T.astype(jnp.int32)
    x5 = _gather_rows(idx_t, table, batch, hist)
    z = jnp.transpose(x5, (0, 1, 3, 2, 4)).reshape(hist, EMBED_DIM, batch)
    return jnp.transpose(z, (2, 0, 1))
